# Initial kernel scaffold; baseline (speedup 1.0000x reference)
import jax
import jax.numpy as jnp
from jax.experimental import pallas as pl


def kernel(x, embedding, pos_embedding):
    raise NotImplementedError("write your pallas kernel here")


# speedup vs baseline: 1.3962x; 1.3962x over previous
import functools

import jax
import jax.numpy as jnp
from jax import lax
from jax.experimental import pallas as pl
from jax.experimental.pallas import tpu as pltpu
from jax.experimental.pallas import tpu_sc as plsc

B = 4096
S = 200
D = 32
TOK = B * S
NC = 2
NS = 16
NW = NC * NS
TOK_PER_W = TOK // NW
SEQ_PER_CHUNK = 4
CHUNK = SEQ_PER_CHUNK * S
NCHUNK = TOK_PER_W // CHUNK


@functools.partial(
    pl.kernel,
    out_type=jax.ShapeDtypeStruct((TOK, D), jnp.float32),
    mesh=plsc.VectorSubcoreMesh(core_axis_name="c", subcore_axis_name="s"),
    scratch_types=[
        pltpu.VMEM((CHUNK,), jnp.int32),
        pltpu.VMEM((CHUNK, D), jnp.float32),
        pltpu.VMEM((S, D), jnp.float32),
        pltpu.SemaphoreType.DMA,
    ],
    compiler_params=pltpu.CompilerParams(use_tc_tiling_on_sc=False),
)
def _emb_lookup(x_hbm, emb_hbm, pos_hbm, out_hbm, idx_v, rows_v, pos_v, sem):
    wid = lax.axis_index("s") * NC + lax.axis_index("c")
    base = wid * TOK_PER_W
    pltpu.sync_copy(pos_hbm, pos_v)

    def chunk_body(c, _):
        off = base + c * CHUNK
        pltpu.sync_copy(x_hbm.at[pl.ds(off, CHUNK)], idx_v)
        pltpu.async_copy(emb_hbm.at[idx_v], rows_v, sem).wait()

        def row_body(r, _):
            p0 = pos_v[r, pl.ds(0, 16)]
            p1 = pos_v[r, pl.ds(16, 16)]
            for sq in range(SEQ_PER_CHUNK):
                rr = sq * S + r
                rows_v[rr, pl.ds(0, 16)] = rows_v[rr, pl.ds(0, 16)] + p0
                rows_v[rr, pl.ds(16, 16)] = rows_v[rr, pl.ds(16, 16)] + p1
            return 0

        lax.fori_loop(0, S, row_body, 0)
        pltpu.sync_copy(rows_v, out_hbm.at[pl.ds(off, CHUNK)])
        return 0

    lax.fori_loop(0, NCHUNK, chunk_body, 0)


def kernel(x, embedding, pos_embedding):
    out = _emb_lookup(x.reshape(TOK), embedding, pos_embedding)
    return out.reshape(B, S, D)
